# baseline (speedup 1.0000x reference)
import jax
import jax.numpy as jnp
from jax.experimental import pallas as pl
from jax.experimental.pallas import tpu as pltpu

DIM = 2048
ND = 8
RK = 64
NTOK = 16384
BLK = 512
NBLK = NTOK // BLK
NDRK = ND * RK


def _proj_block_kernel(ids_ref, x_ref, vall_ref, uallt_ref, s_ref, o_ref):
    x = x_ref[...]
    ids = ids_ref[0, 0, :]
    xb = x.astype(jnp.bfloat16)
    z = jnp.dot(xb, vall_ref[...], preferred_element_type=jnp.float32)
    col_dom = jax.lax.broadcasted_iota(jnp.int32, (BLK, NDRK), 1) // RK
    keep = (ids[:, None] == col_dom).astype(jnp.float32)
    z = z * keep * s_ref[...]
    proj = jnp.dot(z.astype(jnp.bfloat16), uallt_ref[...],
                   preferred_element_type=jnp.float32)
    o_ref[...] = x + proj


def _reg_kernel(ids_ref, u_ref, v_ref, s_ref, reg_ref):
    ids = ids_ref[...].astype(jnp.int32)
    row_i = jax.lax.broadcasted_iota(jnp.int32, (RK, RK), 0)
    col_i = jax.lax.broadcasted_iota(jnp.int32, (RK, RK), 1)
    eye = (row_i == col_i).astype(jnp.float32)
    total = jnp.zeros((), dtype=jnp.float32)
    for d in range(ND):
        cnt = jnp.sum((ids == d).astype(jnp.float32))
        any_d = (cnt > 0).astype(jnp.float32)
        ud = u_ref[d]
        vd = v_ref[d]
        gu = jax.lax.dot_general(ud, ud, (((0,), (0,)), ((), ())),
                                 preferred_element_type=jnp.float32)
        gv = jax.lax.dot_general(vd, vd, (((0,), (0,)), ((), ())),
                                 preferred_element_type=jnp.float32)
        reg_d = jnp.mean((gu - eye) ** 2) + jnp.mean((gv - eye) ** 2)
        reg_d = reg_d + 0.1 * jnp.mean(jnp.abs(s_ref[d]))
        total = total + any_d * reg_d
    reg_ref[...] = jnp.full((8, 128), total / ND, dtype=jnp.float32)


def kernel(feats, domain_ids, U, V, s):
    ids3d = domain_ids.reshape(NBLK, 1, BLK)
    v_all = V.transpose(1, 0, 2).reshape(DIM, NDRK).astype(jnp.bfloat16)
    u_all_t = U.transpose(0, 2, 1).reshape(NDRK, DIM).astype(jnp.bfloat16)
    s_flat = s.reshape(1, NDRK)

    out = pl.pallas_call(
        _proj_block_kernel,
        grid=(NBLK,),
        in_specs=[
            pl.BlockSpec((1, 1, BLK), lambda i: (i, 0, 0)),
            pl.BlockSpec((BLK, DIM), lambda i: (i, 0)),
            pl.BlockSpec((DIM, NDRK), lambda i: (0, 0)),
            pl.BlockSpec((NDRK, DIM), lambda i: (0, 0)),
            pl.BlockSpec((1, NDRK), lambda i: (0, 0)),
        ],
        out_specs=pl.BlockSpec((BLK, DIM), lambda i: (i, 0)),
        out_shape=jax.ShapeDtypeStruct((NTOK, DIM), jnp.float32),
    )(ids3d, feats, v_all, u_all_t, s_flat)

    reg = pl.pallas_call(
        _reg_kernel,
        in_specs=[
            pl.BlockSpec((NTOK, 1), lambda: (0, 0)),
            pl.BlockSpec((ND, DIM, RK), lambda: (0, 0, 0)),
            pl.BlockSpec((ND, DIM, RK), lambda: (0, 0, 0)),
            pl.BlockSpec((ND, RK), lambda: (0, 0)),
        ],
        out_specs=pl.BlockSpec((8, 128), lambda: (0, 0)),
        out_shape=jax.ShapeDtypeStruct((8, 128), jnp.float32),
    )(domain_ids.reshape(NTOK, 1), U, V, s)

    return out, reg[0, 0].reshape(1)


# speedup vs baseline: 1.0221x; 1.0221x over previous
import jax
import jax.numpy as jnp
from jax.experimental import pallas as pl
from jax.experimental.pallas import tpu as pltpu

DIM = 2048
ND = 8
RK = 64
NTOK = 16384
BLK = 512
NBLK = NTOK // BLK
NDRK = ND * RK


def _proj_block_kernel(ids_ref, x_ref, vall_ref, uallt_ref, s_ref, o_ref):
    x = x_ref[...]
    ids = ids_ref[0, 0, :]
    z = jnp.dot(x, vall_ref[...], preferred_element_type=jnp.float32)
    col_dom = jax.lax.broadcasted_iota(jnp.int32, (BLK, NDRK), 1) // RK
    keep = (ids[:, None] == col_dom).astype(jnp.float32)
    z = z * keep * s_ref[...]
    proj = jnp.dot(z, uallt_ref[...], preferred_element_type=jnp.float32)
    o_ref[...] = x + proj


def _reg_kernel(ids_ref, u_ref, v_ref, s_ref, reg_ref):
    ids = ids_ref[...].astype(jnp.int32)
    row_i = jax.lax.broadcasted_iota(jnp.int32, (RK, RK), 0)
    col_i = jax.lax.broadcasted_iota(jnp.int32, (RK, RK), 1)
    eye = (row_i == col_i).astype(jnp.float32)
    total = jnp.zeros((), dtype=jnp.float32)
    for d in range(ND):
        cnt = jnp.sum((ids == d).astype(jnp.float32))
        any_d = (cnt > 0).astype(jnp.float32)
        ud = u_ref[d]
        vd = v_ref[d]
        gu = jax.lax.dot_general(ud, ud, (((0,), (0,)), ((), ())),
                                 preferred_element_type=jnp.float32)
        gv = jax.lax.dot_general(vd, vd, (((0,), (0,)), ((), ())),
                                 preferred_element_type=jnp.float32)
        reg_d = jnp.mean((gu - eye) ** 2) + jnp.mean((gv - eye) ** 2)
        reg_d = reg_d + 0.1 * jnp.mean(jnp.abs(s_ref[d]))
        total = total + any_d * reg_d
    reg_ref[...] = jnp.full((8, 128), total / ND, dtype=jnp.float32)


def kernel(feats, domain_ids, U, V, s):
    ids3d = domain_ids.reshape(NBLK, 1, BLK)
    v_all = V.transpose(1, 0, 2).reshape(DIM, NDRK)
    u_all_t = U.transpose(0, 2, 1).reshape(NDRK, DIM)
    s_flat = s.reshape(1, NDRK)

    out = pl.pallas_call(
        _proj_block_kernel,
        grid=(NBLK,),
        in_specs=[
            pl.BlockSpec((1, 1, BLK), lambda i: (i, 0, 0)),
            pl.BlockSpec((BLK, DIM), lambda i: (i, 0)),
            pl.BlockSpec((DIM, NDRK), lambda i: (0, 0)),
            pl.BlockSpec((NDRK, DIM), lambda i: (0, 0)),
            pl.BlockSpec((1, NDRK), lambda i: (0, 0)),
        ],
        out_specs=pl.BlockSpec((BLK, DIM), lambda i: (i, 0)),
        out_shape=jax.ShapeDtypeStruct((NTOK, DIM), jnp.float32),
    )(ids3d, feats, v_all, u_all_t, s_flat)

    reg = pl.pallas_call(
        _reg_kernel,
        in_specs=[
            pl.BlockSpec((NTOK, 1), lambda: (0, 0)),
            pl.BlockSpec((ND, DIM, RK), lambda: (0, 0, 0)),
            pl.BlockSpec((ND, DIM, RK), lambda: (0, 0, 0)),
            pl.BlockSpec((ND, RK), lambda: (0, 0)),
        ],
        out_specs=pl.BlockSpec((8, 128), lambda: (0, 0)),
        out_shape=jax.ShapeDtypeStruct((8, 128), jnp.float32),
    )(domain_ids.reshape(NTOK, 1), U, V, s)

    return out, reg[0, 0].reshape(1)


# speedup vs baseline: 1.1380x; 1.1134x over previous
import jax
import jax.numpy as jnp
from jax.experimental import pallas as pl
from jax.experimental.pallas import tpu as pltpu

DIM = 2048
ND = 8
RK = 64
NTOK = 16384
BLK = 512
NBLK = NTOK // BLK
NDRK = ND * RK


def _proj_block_kernel(ids_ref, x_ref, vall_ref, uallt_ref, s_ref, o_ref):
    x = x_ref[...]
    ids = ids_ref[0, 0, :]
    z = jnp.dot(x, vall_ref[...], preferred_element_type=jnp.float32)
    col_dom = jax.lax.broadcasted_iota(jnp.int32, (BLK, NDRK), 1) // RK
    keep = (ids[:, None] == col_dom).astype(jnp.float32)
    z = z * keep * s_ref[...]
    proj = jnp.dot(z, uallt_ref[...], preferred_element_type=jnp.float32)
    o_ref[...] = x + proj


def _reg_kernel(ids_ref, u_ref, v_ref, s_ref, reg_ref):
    ids = ids_ref[...].astype(jnp.int32)
    row_i = jax.lax.broadcasted_iota(jnp.int32, (RK, RK), 0)
    col_i = jax.lax.broadcasted_iota(jnp.int32, (RK, RK), 1)
    eye = (row_i == col_i).astype(jnp.float32)
    total = jnp.zeros((), dtype=jnp.float32)
    for d in range(ND):
        cnt = jnp.sum((ids == d).astype(jnp.float32))
        any_d = (cnt > 0).astype(jnp.float32)
        ud = u_ref[d]
        vd = v_ref[d]
        gu = jax.lax.dot_general(ud, ud, (((0,), (0,)), ((), ())),
                                 preferred_element_type=jnp.float32)
        gv = jax.lax.dot_general(vd, vd, (((0,), (0,)), ((), ())),
                                 preferred_element_type=jnp.float32)
        reg_d = jnp.mean((gu - eye) ** 2) + jnp.mean((gv - eye) ** 2)
        reg_d = reg_d + 0.1 * jnp.mean(jnp.abs(s_ref[d]))
        total = total + any_d * reg_d
    reg_ref[...] = jnp.full((8, 128), total / ND, dtype=jnp.float32)


def kernel(feats, domain_ids, U, V, s):
    ids3d = domain_ids.reshape(NBLK, 1, BLK)
    v_all = V.transpose(1, 0, 2).reshape(DIM, NDRK)
    u_all_t = U.transpose(0, 2, 1).reshape(NDRK, DIM)
    s_flat = s.reshape(1, NDRK)

    out = pl.pallas_call(
        _proj_block_kernel,
        grid=(NBLK,),
        in_specs=[
            pl.BlockSpec((1, 1, BLK), lambda i: (i, 0, 0)),
            pl.BlockSpec((BLK, DIM), lambda i: (i, 0)),
            pl.BlockSpec((DIM, NDRK), lambda i: (0, 0)),
            pl.BlockSpec((NDRK, DIM), lambda i: (0, 0)),
            pl.BlockSpec((1, NDRK), lambda i: (0, 0)),
        ],
        out_specs=pl.BlockSpec((BLK, DIM), lambda i: (i, 0)),
        out_shape=jax.ShapeDtypeStruct((NTOK, DIM), jnp.float32),
    )(ids3d, feats, v_all, u_all_t, s_flat)

    reg = pl.pallas_call(
        _reg_kernel,
        in_specs=[
            pl.BlockSpec((128, 128), lambda: (0, 0)),
            pl.BlockSpec((ND, DIM, RK), lambda: (0, 0, 0)),
            pl.BlockSpec((ND, DIM, RK), lambda: (0, 0, 0)),
            pl.BlockSpec((ND, RK), lambda: (0, 0)),
        ],
        out_specs=pl.BlockSpec((8, 128), lambda: (0, 0)),
        out_shape=jax.ShapeDtypeStruct((8, 128), jnp.float32),
    )(domain_ids.reshape(128, 128), U, V, s)

    return out, reg[0, 0].reshape(1)


# speedup vs baseline: 1.2200x; 1.0721x over previous
import jax
import jax.numpy as jnp
from jax.experimental import pallas as pl
from jax.experimental.pallas import tpu as pltpu

DIM = 2048
ND = 8
RK = 64
NTOK = 16384
BLK = 1024
NBLK = NTOK // BLK
NDRK = ND * RK


def _proj_block_kernel(ids_ref, x_ref, vall_ref, uallt_ref, s_ref, o_ref):
    x = x_ref[...]
    ids = ids_ref[0, 0, :]
    z = jnp.dot(x, vall_ref[...], preferred_element_type=jnp.float32)
    col_dom = jax.lax.broadcasted_iota(jnp.int32, (BLK, NDRK), 1) // RK
    keep = (ids[:, None] == col_dom).astype(jnp.float32)
    z = z * keep * s_ref[...]
    proj = jnp.dot(z, uallt_ref[...], preferred_element_type=jnp.float32)
    o_ref[...] = x + proj


def _reg_kernel(ids_ref, u_ref, v_ref, s_ref, reg_ref):
    ids = ids_ref[...].astype(jnp.int32)
    row_i = jax.lax.broadcasted_iota(jnp.int32, (RK, RK), 0)
    col_i = jax.lax.broadcasted_iota(jnp.int32, (RK, RK), 1)
    eye = (row_i == col_i).astype(jnp.float32)
    total = jnp.zeros((), dtype=jnp.float32)
    for d in range(ND):
        cnt = jnp.sum((ids == d).astype(jnp.float32))
        any_d = (cnt > 0).astype(jnp.float32)
        ud = u_ref[d]
        vd = v_ref[d]
        gu = jax.lax.dot_general(ud, ud, (((0,), (0,)), ((), ())),
                                 preferred_element_type=jnp.float32)
        gv = jax.lax.dot_general(vd, vd, (((0,), (0,)), ((), ())),
                                 preferred_element_type=jnp.float32)
        reg_d = jnp.mean((gu - eye) ** 2) + jnp.mean((gv - eye) ** 2)
        reg_d = reg_d + 0.1 * jnp.mean(jnp.abs(s_ref[d]))
        total = total + any_d * reg_d
    reg_ref[...] = jnp.full((8, 128), total / ND, dtype=jnp.float32)


def kernel(feats, domain_ids, U, V, s):
    ids3d = domain_ids.reshape(NBLK, 1, BLK)
    v_all = V.transpose(1, 0, 2).reshape(DIM, NDRK)
    u_all_t = U.transpose(0, 2, 1).reshape(NDRK, DIM)
    s_flat = s.reshape(1, NDRK)

    out = pl.pallas_call(
        _proj_block_kernel,
        grid=(NBLK,),
        in_specs=[
            pl.BlockSpec((1, 1, BLK), lambda i: (i, 0, 0)),
            pl.BlockSpec((BLK, DIM), lambda i: (i, 0)),
            pl.BlockSpec((DIM, NDRK), lambda i: (0, 0)),
            pl.BlockSpec((NDRK, DIM), lambda i: (0, 0)),
            pl.BlockSpec((1, NDRK), lambda i: (0, 0)),
        ],
        out_specs=pl.BlockSpec((BLK, DIM), lambda i: (i, 0)),
        out_shape=jax.ShapeDtypeStruct((NTOK, DIM), jnp.float32),
    )(ids3d, feats, v_all, u_all_t, s_flat)

    reg = pl.pallas_call(
        _reg_kernel,
        in_specs=[
            pl.BlockSpec((128, 128), lambda: (0, 0)),
            pl.BlockSpec((ND, DIM, RK), lambda: (0, 0, 0)),
            pl.BlockSpec((ND, DIM, RK), lambda: (0, 0, 0)),
            pl.BlockSpec((ND, RK), lambda: (0, 0)),
        ],
        out_specs=pl.BlockSpec((8, 128), lambda: (0, 0)),
        out_shape=jax.ShapeDtypeStruct((8, 128), jnp.float32),
    )(domain_ids.reshape(128, 128), U, V, s)

    return out, reg[0, 0].reshape(1)
